# Initial kernel scaffold; baseline (speedup 1.0000x reference)
import jax
import jax.numpy as jnp
from jax.experimental import pallas as pl


def kernel(rpn_cls_prob, rpn_bbox_pred, im_info, anchors):
    raise NotImplementedError("write your pallas kernel here")


# speedup vs baseline: 11.3970x; 11.3970x over previous
import functools

import jax
import jax.numpy as jnp
from jax.experimental import pallas as pl

_PRE_NMS_TOP_N = 2000
_POST_NMS_TOP_N = 300
_NMS_THRESH = 0.7
_MIN_SIZE = 4.0
_FEAT_STRIDE = 8.0
_PAD = 2048


def _proposal_nms_kernel(order_ref, deltas_ref, anchors_ref, im_info_ref,
                         keep_ref, props_ref, *, num_anchors, width, height):
    idx = jax.lax.broadcasted_iota(jnp.int32, (1, _PAD), 1)
    order = order_ref[0:1, :]

    a = order % num_anchors
    cell = order // num_anchors
    xg = (cell % width).astype(jnp.float32) * _FEAT_STRIDE
    yg = ((cell // width) % height).astype(jnp.float32) * _FEAT_STRIDE
    zg = (cell // (width * height)).astype(jnp.float32) * _FEAT_STRIDE

    def anchor_col(col):
        v = anchors_ref[0, col]
        for r in range(1, num_anchors):
            v = jnp.where(a == r, anchors_ref[r, col], v)
        return v

    ax1 = anchor_col(0) + xg
    ay1 = anchor_col(1) + yg
    az1 = anchor_col(2) + zg
    ax2 = anchor_col(3) + xg
    ay2 = anchor_col(4) + yg
    az2 = anchor_col(5) + zg

    ws = ax2 - ax1 + 1.0
    hs = ay2 - ay1 + 1.0
    ds = az2 - az1 + 1.0
    cx = ax1 + 0.5 * ws
    cy = ay1 + 0.5 * hs
    cz = az1 + 0.5 * ds

    dx = deltas_ref[0:1, :]
    dy = deltas_ref[1:2, :]
    dz = deltas_ref[2:3, :]
    dw = deltas_ref[3:4, :]
    dh = deltas_ref[4:5, :]
    dd = deltas_ref[5:6, :]

    pcx = dx * ws + cx
    pcy = dy * hs + cy
    pcz = dz * ds + cz
    pw = jnp.exp(dw) * ws
    ph = jnp.exp(dh) * hs
    pd = jnp.exp(dd) * ds

    x1 = pcx - 0.5 * pw
    y1 = pcy - 0.5 * ph
    z1 = pcz - 0.5 * pd
    x2 = pcx + 0.5 * pw - 1.0
    y2 = pcy + 0.5 * ph - 1.0
    z2 = pcz + 0.5 * pd - 1.0

    slices = im_info_ref[0, 0]
    im_h = im_info_ref[0, 1]
    im_w = im_info_ref[0, 2]
    scale = im_info_ref[0, 3]

    x1 = jnp.clip(x1, 0.0, im_w - 1.0)
    y1 = jnp.clip(y1, 0.0, im_h - 1.0)
    z1 = jnp.clip(z1, 0.0, slices - 1.0)
    x2 = jnp.clip(x2, 0.0, im_w - 1.0)
    y2 = jnp.clip(y2, 0.0, im_h - 1.0)
    z2 = jnp.clip(z2, 0.0, slices - 1.0)

    ms = _MIN_SIZE * scale
    ss = x2 - x1 + 1.0
    xc = x1 + ss / 2.0
    yc = y1 + ss / 2.0
    zc = z1 + ss / 2.0
    fmask = (ss >= ms) & (xc < im_w) & (yc < im_h) & (zc < slices)
    fmask = fmask & (idx < _PRE_NMS_TOP_N)

    w = jnp.maximum(x2 - x1 + 1.0, 0.0)
    h = jnp.maximum(y2 - y1 + 1.0, 0.0)
    d = jnp.maximum(z2 - z1 + 1.0, 0.0)
    vol = w * h * d

    def body(i, keep):
        eq = idx == i
        ki = jnp.sum(jnp.where(eq, keep, 0.0)) > 0.0
        gx1 = jnp.sum(jnp.where(eq, x1, 0.0))
        gy1 = jnp.sum(jnp.where(eq, y1, 0.0))
        gz1 = jnp.sum(jnp.where(eq, z1, 0.0))
        gx2 = jnp.sum(jnp.where(eq, x2, 0.0))
        gy2 = jnp.sum(jnp.where(eq, y2, 0.0))
        gz2 = jnp.sum(jnp.where(eq, z2, 0.0))
        gv = jnp.sum(jnp.where(eq, vol, 0.0))
        iw = jnp.maximum(jnp.minimum(x2, gx2) - jnp.maximum(x1, gx1) + 1.0, 0.0)
        ih = jnp.maximum(jnp.minimum(y2, gy2) - jnp.maximum(y1, gy1) + 1.0, 0.0)
        idp = jnp.maximum(jnp.minimum(z2, gz2) - jnp.maximum(z1, gz1) + 1.0, 0.0)
        inter = iw * ih * idp
        iou = inter / (vol + gv - inter + 1e-8)
        sup = ki & (idx > i) & (iou > _NMS_THRESH)
        return jnp.where(sup, 0.0, keep)

    keep = jax.lax.fori_loop(0, _PRE_NMS_TOP_N, body,
                             fmask.astype(jnp.float32))

    keep_ref[0:1, :] = keep
    props_ref[0:1, :] = x1
    props_ref[1:2, :] = y1
    props_ref[2:3, :] = z1
    props_ref[3:4, :] = x2
    props_ref[4:5, :] = y2
    props_ref[5:6, :] = z2


def _proposals_one_image(im_info_i, anchors, bbox_deltas, scores,
                         num_anchors, S, H, W):
    sc = jnp.transpose(scores, (1, 2, 3, 0)).reshape(-1)
    top_scores, order = jax.lax.top_k(sc, _PRE_NMS_TOP_N)

    d_all = jnp.transpose(bbox_deltas, (1, 2, 3, 0)).reshape(-1, 6)
    d_g = jnp.take(d_all, order, axis=0)
    d_in = jnp.zeros((6, _PAD), jnp.float32).at[:, :_PRE_NMS_TOP_N].set(d_g.T)
    order_in = jnp.zeros((1, _PAD), jnp.int32).at[0, :_PRE_NMS_TOP_N].set(order)
    im_in = im_info_i.reshape(1, 4).astype(jnp.float32)

    kern = functools.partial(_proposal_nms_kernel, num_anchors=num_anchors,
                             width=W, height=H)
    keep_f, props = pl.pallas_call(
        kern,
        out_shape=(
            jax.ShapeDtypeStruct((1, _PAD), jnp.float32),
            jax.ShapeDtypeStruct((6, _PAD), jnp.float32),
        ),
    )(order_in, d_in, anchors.astype(jnp.float32), im_in)

    keep = keep_f[0, :_PRE_NMS_TOP_N] > 0.0
    final_scores = jnp.where(keep, top_scores, -jnp.inf)
    sel_scores, sel = jax.lax.top_k(final_scores, _POST_NMS_TOP_N)
    out_valid = jnp.isfinite(sel_scores)
    boxes = jnp.take(props[:, :_PRE_NMS_TOP_N], sel, axis=1).T
    out_boxes = jnp.where(out_valid[:, None], boxes, 0.0)
    out_scores = jnp.where(out_valid, sel_scores, 0.0)[:, None]
    out_idx = jnp.where(out_valid, jnp.take(order, sel), -1)
    return out_boxes, out_scores, out_idx


def kernel(rpn_cls_prob, rpn_bbox_pred, im_info, anchors):
    num_images = rpn_cls_prob.shape[0]
    num_anchors = rpn_cls_prob.shape[1]
    S, H, W = rpn_cls_prob.shape[-3:]
    rois_list, probs_list, idx_list = [], [], []
    for i in range(num_images):
        b, p, ix = _proposals_one_image(im_info[i], anchors,
                                        rpn_bbox_pred[i], rpn_cls_prob[i],
                                        num_anchors, S, H, W)
        batch_inds = jnp.full((b.shape[0], 1), float(i), dtype=b.dtype)
        rois_list.append(jnp.concatenate([batch_inds, b], axis=1))
        probs_list.append(p)
        idx_list.append(ix)
    rois = jnp.concatenate(rois_list, axis=0)
    roi_probs = jnp.concatenate(probs_list, axis=0)
    keep_idx = jnp.concatenate(idx_list, axis=0)
    return rois, roi_probs, keep_idx


# speedup vs baseline: 14.5518x; 1.2768x over previous
import functools

import jax
import jax.numpy as jnp
from jax.experimental import pallas as pl

_PRE_NMS_TOP_N = 2000
_POST_NMS_TOP_N = 300
_NMS_THRESH = 0.7
_MIN_SIZE = 4.0
_FEAT_STRIDE = 8.0
_SUB = 16
_LANE = 128
_PAD = _SUB * _LANE


def _proposal_nms_kernel(order_ref, deltas_ref, anchors_ref, im_info_ref,
                         keep_ref, props_ref, *, num_anchors, width, height):
    shape = (_SUB, _LANE)
    idx = (jax.lax.broadcasted_iota(jnp.int32, shape, 0) * _LANE
           + jax.lax.broadcasted_iota(jnp.int32, shape, 1))
    order = order_ref[...]

    a = order % num_anchors
    cell = order // num_anchors
    xg = (cell % width).astype(jnp.float32) * _FEAT_STRIDE
    yg = ((cell // width) % height).astype(jnp.float32) * _FEAT_STRIDE
    zg = (cell // (width * height)).astype(jnp.float32) * _FEAT_STRIDE

    def anchor_col(col):
        v = anchors_ref[0, col]
        for r in range(1, num_anchors):
            v = jnp.where(a == r, anchors_ref[r, col], v)
        return v

    ax1 = anchor_col(0) + xg
    ay1 = anchor_col(1) + yg
    az1 = anchor_col(2) + zg
    ax2 = anchor_col(3) + xg
    ay2 = anchor_col(4) + yg
    az2 = anchor_col(5) + zg

    ws = ax2 - ax1 + 1.0
    hs = ay2 - ay1 + 1.0
    ds = az2 - az1 + 1.0
    cx = ax1 + 0.5 * ws
    cy = ay1 + 0.5 * hs
    cz = az1 + 0.5 * ds

    dx = deltas_ref[0]
    dy = deltas_ref[1]
    dz = deltas_ref[2]
    dw = deltas_ref[3]
    dh = deltas_ref[4]
    dd = deltas_ref[5]

    pcx = dx * ws + cx
    pcy = dy * hs + cy
    pcz = dz * ds + cz
    pw = jnp.exp(dw) * ws
    ph = jnp.exp(dh) * hs
    pd = jnp.exp(dd) * ds

    x1 = pcx - 0.5 * pw
    y1 = pcy - 0.5 * ph
    z1 = pcz - 0.5 * pd
    x2 = pcx + 0.5 * pw - 1.0
    y2 = pcy + 0.5 * ph - 1.0
    z2 = pcz + 0.5 * pd - 1.0

    slices = im_info_ref[0, 0]
    im_h = im_info_ref[0, 1]
    im_w = im_info_ref[0, 2]
    scale = im_info_ref[0, 3]

    x1 = jnp.clip(x1, 0.0, im_w - 1.0)
    y1 = jnp.clip(y1, 0.0, im_h - 1.0)
    z1 = jnp.clip(z1, 0.0, slices - 1.0)
    x2 = jnp.clip(x2, 0.0, im_w - 1.0)
    y2 = jnp.clip(y2, 0.0, im_h - 1.0)
    z2 = jnp.clip(z2, 0.0, slices - 1.0)

    ms = _MIN_SIZE * scale
    ss = x2 - x1 + 1.0
    xc = x1 + ss / 2.0
    yc = y1 + ss / 2.0
    zc = z1 + ss / 2.0
    fmask = (ss >= ms) & (xc < im_w) & (yc < im_h) & (zc < slices)
    fmask = fmask & (idx < _PRE_NMS_TOP_N)

    w = jnp.maximum(x2 - x1 + 1.0, 0.0)
    h = jnp.maximum(y2 - y1 + 1.0, 0.0)
    d = jnp.maximum(z2 - z1 + 1.0, 0.0)
    vol = w * h * d

    def body(i, keep):
        eq = idx == i
        ki = jnp.sum(jnp.where(eq, keep, 0.0)) > 0.0
        gx1 = jnp.sum(jnp.where(eq, x1, 0.0))
        gy1 = jnp.sum(jnp.where(eq, y1, 0.0))
        gz1 = jnp.sum(jnp.where(eq, z1, 0.0))
        gx2 = jnp.sum(jnp.where(eq, x2, 0.0))
        gy2 = jnp.sum(jnp.where(eq, y2, 0.0))
        gz2 = jnp.sum(jnp.where(eq, z2, 0.0))
        gv = jnp.sum(jnp.where(eq, vol, 0.0))
        iw = jnp.maximum(jnp.minimum(x2, gx2) - jnp.maximum(x1, gx1) + 1.0, 0.0)
        ih = jnp.maximum(jnp.minimum(y2, gy2) - jnp.maximum(y1, gy1) + 1.0, 0.0)
        idp = jnp.maximum(jnp.minimum(z2, gz2) - jnp.maximum(z1, gz1) + 1.0, 0.0)
        inter = iw * ih * idp
        iou = inter / (vol + gv - inter + 1e-8)
        sup = ki & (idx > i) & (iou > _NMS_THRESH)
        return jnp.where(sup, 0.0, keep)

    keep = jax.lax.fori_loop(0, _PRE_NMS_TOP_N, body,
                             fmask.astype(jnp.float32))

    keep_ref[...] = keep
    props_ref[0] = x1
    props_ref[1] = y1
    props_ref[2] = z1
    props_ref[3] = x2
    props_ref[4] = y2
    props_ref[5] = z2


def _proposals_one_image(im_info_i, anchors, bbox_deltas, scores,
                         num_anchors, S, H, W):
    sc = jnp.transpose(scores, (1, 2, 3, 0)).reshape(-1)
    top_scores, order = jax.lax.top_k(sc, _PRE_NMS_TOP_N)

    d_all = jnp.transpose(bbox_deltas, (1, 2, 3, 0)).reshape(-1, 6)
    d_g = jnp.take(d_all, order, axis=0)
    d_in = (jnp.zeros((6, _PAD), jnp.float32)
            .at[:, :_PRE_NMS_TOP_N].set(d_g.T)
            .reshape(6, _SUB, _LANE))
    order_in = (jnp.zeros((_PAD,), jnp.int32)
                .at[:_PRE_NMS_TOP_N].set(order)
                .reshape(_SUB, _LANE))
    im_in = im_info_i.reshape(1, 4).astype(jnp.float32)

    kern = functools.partial(_proposal_nms_kernel, num_anchors=num_anchors,
                             width=W, height=H)
    keep_f, props = pl.pallas_call(
        kern,
        out_shape=(
            jax.ShapeDtypeStruct((_SUB, _LANE), jnp.float32),
            jax.ShapeDtypeStruct((6, _SUB, _LANE), jnp.float32),
        ),
    )(order_in, d_in, anchors.astype(jnp.float32), im_in)

    keep = keep_f.reshape(_PAD)[:_PRE_NMS_TOP_N] > 0.0
    final_scores = jnp.where(keep, top_scores, -jnp.inf)
    sel_scores, sel = jax.lax.top_k(final_scores, _POST_NMS_TOP_N)
    out_valid = jnp.isfinite(sel_scores)
    props2 = props.reshape(6, _PAD)[:, :_PRE_NMS_TOP_N]
    boxes = jnp.take(props2, sel, axis=1).T
    out_boxes = jnp.where(out_valid[:, None], boxes, 0.0)
    out_scores = jnp.where(out_valid, sel_scores, 0.0)[:, None]
    out_idx = jnp.where(out_valid, jnp.take(order, sel), -1)
    return out_boxes, out_scores, out_idx


def kernel(rpn_cls_prob, rpn_bbox_pred, im_info, anchors):
    num_images = rpn_cls_prob.shape[0]
    num_anchors = rpn_cls_prob.shape[1]
    S, H, W = rpn_cls_prob.shape[-3:]
    rois_list, probs_list, idx_list = [], [], []
    for i in range(num_images):
        b, p, ix = _proposals_one_image(im_info[i], anchors,
                                        rpn_bbox_pred[i], rpn_cls_prob[i],
                                        num_anchors, S, H, W)
        batch_inds = jnp.full((b.shape[0], 1), float(i), dtype=b.dtype)
        rois_list.append(jnp.concatenate([batch_inds, b], axis=1))
        probs_list.append(p)
        idx_list.append(ix)
    rois = jnp.concatenate(rois_list, axis=0)
    roi_probs = jnp.concatenate(probs_list, axis=0)
    keep_idx = jnp.concatenate(idx_list, axis=0)
    return rois, roi_probs, keep_idx
